# baseline (speedup 1.0000x reference)
import jax
import jax.numpy as jnp
from jax.experimental import pallas as pl

D = 768
NUM_EXPERTS = 64
TOP_K = 2
ROUTER_DIM = 16
TOKENS = 32768

BLOCK = 2048
HALF = TOKENS // 2


def _top2_softmax(scores):
    eidx = jax.lax.broadcasted_iota(jnp.int32, scores.shape, 1)
    m1 = jnp.max(scores, axis=1, keepdims=True)
    i1 = jnp.min(jnp.where(scores == m1, eidx, NUM_EXPERTS),
                 axis=1, keepdims=True)
    masked = jnp.where(eidx == i1, -jnp.inf, scores)
    m2 = jnp.max(masked, axis=1, keepdims=True)
    i2 = jnp.min(jnp.where(masked == m2, eidx, NUM_EXPERTS),
                 axis=1, keepdims=True)
    idx = jnp.concatenate([i1, i2], axis=1)
    e = jnp.exp(m2 - m1)
    denom = 1.0 + e
    probs = jnp.concatenate([1.0 / denom, e / denom], axis=1)
    return idx, probs


def _router_block(xa_ref, xb_ref, wq_ref, keys_ref,
                  idx_ref, probs_ref, scores_ref):
    wq = wq_ref[...]
    keys = keys_ref[...]

    for h, x in enumerate((xa_ref[...], xb_ref[...])):
        q = jax.lax.dot_general(
            x, wq, (((1,), (1,)), ((), ())),
            preferred_element_type=jnp.float32,
        )
        scores = jax.lax.dot_general(
            q, keys, (((1,), (1,)), ((), ())),
            preferred_element_type=jnp.float32,
        )
        scores_ref[h] = scores
        idx, probs = _top2_softmax(scores)
        idx_ref[h] = idx
        probs_ref[h] = probs


@jax.jit
def kernel(x, W_query, keys):
    grid = (HALF // BLOCK,)
    out_types = (
        jax.ShapeDtypeStruct((2, HALF, TOP_K), jnp.int32),
        jax.ShapeDtypeStruct((2, HALF, TOP_K), jnp.float32),
        jax.ShapeDtypeStruct((2, HALF, NUM_EXPERTS), jnp.float32),
    )
    topk_idx, probs, scores = pl.pallas_call(
        _router_block,
        grid=grid,
        in_specs=[
            pl.BlockSpec((BLOCK, D), lambda i: (i, 0)),
            pl.BlockSpec((BLOCK, D), lambda i: (i + HALF // BLOCK, 0)),
            pl.BlockSpec((ROUTER_DIM, D), lambda i: (0, 0)),
            pl.BlockSpec((NUM_EXPERTS, ROUTER_DIM), lambda i: (0, 0)),
        ],
        out_specs=(
            pl.BlockSpec((2, BLOCK, TOP_K), lambda i: (0, i, 0)),
            pl.BlockSpec((2, BLOCK, TOP_K), lambda i: (0, i, 0)),
            pl.BlockSpec((2, BLOCK, NUM_EXPERTS), lambda i: (0, i, 0)),
        ),
        out_shape=out_types,
    )(x, x, W_query, keys)
    return (topk_idx.reshape(TOKENS, TOP_K),
            probs.reshape(TOKENS, TOP_K),
            scores.reshape(TOKENS, NUM_EXPERTS))


# speedup vs baseline: 1.2074x; 1.2074x over previous
import functools

import jax
import jax.numpy as jnp
from jax import lax
from jax.experimental import pallas as pl
from jax.experimental.pallas import tpu as pltpu
from jax.experimental.pallas import tpu_sc as plsc

D = 768
NUM_EXPERTS = 64
TOP_K = 2
ROUTER_DIM = 16
TOKENS = 32768

BLOCK = 4096

NC, NS, L = 2, 16, 16
NW = NC * NS
ROWS_W = TOKENS // NW


def _scores_block(x_ref, wq_ref, keys_ref, scores_ref, scores_t_ref):
    q = jax.lax.dot_general(
        x_ref[...], wq_ref[...], (((1,), (1,)), ((), ())),
        preferred_element_type=jnp.float32,
    )
    scores = jax.lax.dot_general(
        q, keys_ref[...], (((1,), (1,)), ((), ())),
        preferred_element_type=jnp.float32,
    )
    scores_ref[...] = scores
    scores_t_ref[...] = scores.T


def _tc_scores(x, W_query, keys):
    return pl.pallas_call(
        _scores_block,
        grid=(TOKENS // BLOCK,),
        in_specs=[
            pl.BlockSpec((BLOCK, D), lambda i: (i, 0)),
            pl.BlockSpec((ROUTER_DIM, D), lambda i: (0, 0)),
            pl.BlockSpec((NUM_EXPERTS, ROUTER_DIM), lambda i: (0, 0)),
        ],
        out_specs=(
            pl.BlockSpec((BLOCK, NUM_EXPERTS), lambda i: (i, 0)),
            pl.BlockSpec((NUM_EXPERTS, BLOCK), lambda i: (0, i)),
        ),
        out_shape=(
            jax.ShapeDtypeStruct((TOKENS, NUM_EXPERTS), jnp.float32),
            jax.ShapeDtypeStruct((NUM_EXPERTS, TOKENS), jnp.float32),
        ),
    )(x, W_query, keys)


@functools.partial(
    pl.kernel,
    mesh=plsc.VectorSubcoreMesh(core_axis_name="c", subcore_axis_name="s"),
    out_type=(
        jax.ShapeDtypeStruct((TOKENS,), jnp.int32),
        jax.ShapeDtypeStruct((TOKENS,), jnp.int32),
        jax.ShapeDtypeStruct((TOKENS,), jnp.float32),
        jax.ShapeDtypeStruct((TOKENS,), jnp.float32),
    ),
    scratch_types=[
        pltpu.VMEM((NUM_EXPERTS, ROWS_W), jnp.float32),
        pltpu.VMEM((ROWS_W,), jnp.int32),
        pltpu.VMEM((ROWS_W,), jnp.int32),
        pltpu.VMEM((ROWS_W,), jnp.float32),
        pltpu.VMEM((ROWS_W,), jnp.float32),
    ],
)
def _sc_route(scores_t_hbm, i1_hbm, i2_hbm, p1_hbm, p2_hbm,
              sc_t, sc_i1, sc_i2, sc_m1, sc_m2):
    wid = lax.axis_index("s") * NC + lax.axis_index("c")
    row0 = wid * ROWS_W
    pltpu.sync_copy(scores_t_hbm.at[:, pl.ds(row0, ROWS_W)], sc_t)

    ninf = jnp.full((L,), -jnp.inf, jnp.float32)
    zi = jnp.zeros((L,), jnp.int32)

    def group_body(g, carry):
        del carry
        sl = pl.ds(g * L, L)
        m1, m2 = ninf, ninf
        i1, i2 = zi, zi
        for e in range(NUM_EXPERTS):
            v = sc_t[e, sl]
            e_vec = jnp.full((L,), e, jnp.int32)
            gt1 = v > m1
            gt2 = v > m2
            i2 = jnp.where(gt1, i1, jnp.where(gt2, e_vec, i2))
            m2 = jnp.where(gt1, m1, jnp.where(gt2, v, m2))
            i1 = jnp.where(gt1, e_vec, i1)
            m1 = jnp.where(gt1, v, m1)
        ex = jnp.exp(m2 - m1)
        d = 1.0 + ex
        sc_i1[sl] = i1
        sc_i2[sl] = i2
        sc_m1[sl] = 1.0 / d
        sc_m2[sl] = ex / d
        return 0

    lax.fori_loop(0, ROWS_W // L, group_body, 0)

    pltpu.sync_copy(sc_i1, i1_hbm.at[pl.ds(row0, ROWS_W)])
    pltpu.sync_copy(sc_i2, i2_hbm.at[pl.ds(row0, ROWS_W)])
    pltpu.sync_copy(sc_m1, p1_hbm.at[pl.ds(row0, ROWS_W)])
    pltpu.sync_copy(sc_m2, p2_hbm.at[pl.ds(row0, ROWS_W)])


@jax.jit
def kernel(x, W_query, keys):
    scores, scores_t = _tc_scores(x, W_query, keys)
    i1, i2, p1, p2 = _sc_route(scores_t)
    return (jnp.stack([i1, i2], axis=1),
            jnp.stack([p1, p2], axis=1),
            scores)
